# baseline (speedup 1.0000x reference)
import functools

import jax
import jax.numpy as jnp
from jax import lax
from jax.experimental import pallas as pl
from jax.experimental.pallas import tpu as pltpu
from jax.experimental.pallas import tpu_sc as plsc

_B, _L, _N, _S, _H = 16, 512, 16, 5, 64
_NT = _B * _L * _N
_NC, _NS = 2, 16
_NW = _NC * _NS
_NPW = _NT // _NW
_CH = 64
_NCHUNK = _NPW // _CH
_ROWS = _NT // _N
_F = _N * _H


def _sc_body(table, idx_hbm, mask_hbm, out4, idx_v, mask_v, idxc, rows_v,
             out_v, g0, g1, o0, o1):
    gsem = (g0, g1)
    osem = (o0, o1)
    wid = lax.axis_index("s") * _NC + lax.axis_index("c")
    base_e = wid * (_NPW * _S)
    pltpu.sync_copy(idx_hbm.at[pl.ds(base_e, _NPW * _S)], idx_v)
    pltpu.sync_copy(mask_hbm.at[pl.ds(base_e, _NPW * _S)], mask_v)
    iota5 = lax.iota(jnp.int32, 16) * _S

    def fire(j, b):
        for s in range(_S):
            for g in range(_CH // 16):
                off = (j * _CH + g * 16) * _S + s
                idxc[b, s, pl.ds(g * 16, 16)] = plsc.load_gather(
                    idx_v, [iota5 + off])
        for s in range(_S):
            pltpu.async_copy(table.at[idxc.at[b, s]], rows_v.at[b, s],
                             gsem[b])

    def gwait(b):
        for s in range(_S):
            pltpu.make_async_copy(table.at[idxc.at[b, s]],
                                  rows_v.at[b, s], gsem[b]).wait()

    def owait(b):
        pltpu.make_async_copy(out_v.at[b],
                              out4.at[0, :, pl.ds(0, _CH // _N), :],
                              osem[b]).wait()

    def compute(j, b):
        def group(g, carry):
            goff = (j * _CH + g * 16) * _S
            mvs = [plsc.load_gather(mask_v, [iota5 + (goff + s)])
                   for s in range(_S)]
            for ii in range(16):
                ms = [mvs[s][ii] for s in range(_S)]
                i = g * 16 + ii
                for k in range(_H // 16):
                    sl = pl.ds(k * 16, 16)
                    acc = ms[0] * rows_v[b, 0, i, sl]
                    for s in range(1, _S):
                        acc = acc + ms[s] * rows_v[b, s, i, sl]
                    out_v[b, ii // 2, g,
                          pl.ds((ii % 2) * _H + k * 16, 16)] = acc
            return carry
        lax.fori_loop(0, _CH // 16, group, 0)

    fire(0, 0)
    fire(1, 1)

    def step(t, carry):
        for b in range(2):
            j = 2 * t + b
            gwait(b)

            @pl.when(j >= 2)
            def _():
                owait(b)

            compute(j, b)
            rtile = wid * (_NPW // _N // 8) + j // 2
            rsub = (j % 2) * 4
            pltpu.async_copy(out_v.at[b],
                             out4.at[rtile, :, pl.ds(rsub, _CH // _N), :],
                             osem[b])

            @pl.when(j + 2 < _NCHUNK)
            def _():
                fire(j + 2, b)
        return carry

    lax.fori_loop(0, _NCHUNK // 2, step, 0)
    owait(0)
    owait(1)


@functools.cache
def _sc_gather_fn():
    return pl.kernel(
        _sc_body,
        out_type=jax.ShapeDtypeStruct((_ROWS // 8, _F // 128, 8, 128),
                                      jnp.float32),
        mesh=plsc.VectorSubcoreMesh(core_axis_name="c", subcore_axis_name="s",
                                    num_cores=_NC, num_subcores=_NS),
        scratch_types=[
            pltpu.VMEM((_NPW * _S,), jnp.int32),
            pltpu.VMEM((_NPW * _S,), jnp.float32),
            pltpu.VMEM((2, _S, _CH), jnp.int32),
            pltpu.VMEM((2, _S, _CH, _H), jnp.float32),
            pltpu.VMEM((2, _F // 128, _CH // _N, 128), jnp.float32),
            pltpu.SemaphoreType.DMA,
            pltpu.SemaphoreType.DMA,
            pltpu.SemaphoreType.DMA,
            pltpu.SemaphoreType.DMA,
        ],
        compiler_params=pltpu.CompilerParams(use_tc_tiling_on_sc=False,
                                             needs_layout_passes=False),
    )


def _tc_body(x_ref, pc_ref, wl_ref, wp_ref, lw_ref, lb_ref, g_ref, bb_ref,
             o_ref):
    hp = lax.Precision.HIGHEST
    pc = pc_ref[...]
    lw = lw_ref[...]
    wl = wl_ref[...]
    d = wp_ref[...] - wl
    f_id = lax.broadcasted_iota(jnp.int32, (_F, _N), 0)
    n_id = lax.broadcasted_iota(jnp.int32, (_F, _N), 1)
    sel = jnp.where(f_id // _H == n_id, 1.0, 0.0)
    a_t = lax.dot_general(lw * d, sel, (((1,), (0,)), ((), ())),
                          precision=hp, preferred_element_type=jnp.float32)
    c = lax.dot_general(wl, lw, (((1,), (1,)), ((), ())),
                        precision=hp, preferred_element_type=jnp.float32)
    y = lax.dot_general(pc, a_t, (((1,), (1,)), ((), ())),
                        precision=hp, preferred_element_type=jnp.float32)
    for ct in range(_F // 128):
        xc = x_ref[:, ct].reshape(_BR, 128)
        lwc = lw[:, ct * 128:(ct + 1) * 128]
        y = y + lax.dot_general(xc, lwc, (((1,), (1,)), ((), ())),
                                precision=hp,
                                preferred_element_type=jnp.float32)
    y = y + c + lb_ref[...]
    mu = jnp.mean(y, axis=1, keepdims=True)
    yc = y - mu
    var = jnp.mean(yc * yc, axis=1, keepdims=True)
    o_ref[...] = yc * lax.rsqrt(var + 1e-5) * g_ref[...] + bb_ref[...]


_BR = 512

_tc_mix = pl.pallas_call(
    _tc_body,
    out_shape=jax.ShapeDtypeStruct((_ROWS, _H), jnp.float32),
    grid=(_ROWS // _BR,),
    in_specs=[
        pl.BlockSpec((_BR // 8, _F // 128, 8, 128), lambda i: (i, 0, 0, 0)),
        pl.BlockSpec((_BR, _N), lambda i: (i, 0)),
        pl.BlockSpec((1, _F), lambda i: (0, 0)),
        pl.BlockSpec((1, _F), lambda i: (0, 0)),
        pl.BlockSpec((_H, _F), lambda i: (0, 0)),
        pl.BlockSpec((1, _H), lambda i: (0, 0)),
        pl.BlockSpec((1, _H), lambda i: (0, 0)),
        pl.BlockSpec((1, _H), lambda i: (0, 0)),
    ],
    out_specs=pl.BlockSpec((_BR, _H), lambda i: (i, 0)),
)


def kernel(src, pos_coeff, src_subtoken_mask, tok_embedding, W_level,
           W_parent, lin_W, lin_b, ln_g, ln_b):
    x4 = _sc_gather_fn()(tok_embedding, src.reshape(-1),
                         src_subtoken_mask.reshape(-1))
    out = _tc_mix(x4, pos_coeff.reshape(_ROWS, _N),
                  W_level.reshape(1, _F), W_parent.reshape(1, _F), lin_W,
                  lin_b.reshape(1, _H), ln_g.reshape(1, _H),
                  ln_b.reshape(1, _H))
    return out.reshape(_B, _L, _H)


# speedup vs baseline: 1.0338x; 1.0338x over previous
import functools

import jax
import jax.numpy as jnp
from jax import lax
from jax.experimental import pallas as pl
from jax.experimental.pallas import tpu as pltpu
from jax.experimental.pallas import tpu_sc as plsc

_B, _L, _N, _S, _H = 16, 512, 16, 5, 64
_NT = _B * _L * _N
_NC, _NS = 2, 16
_NW = _NC * _NS
_NPW = _NT // _NW
_CH = 64
_NCHUNK = _NPW // _CH
_ROWS = _NT // _N
_F = _N * _H


_CHN = 128
_CE = _CHN * _S
_NCH = _NPW // _CHN
_IR = _CE // 128


def _sc_body(table, idx2, maskf, out4, sidx, smask, rows_v, out_v,
             g0, g1, o0, o1, s0, s1, s2, s3):
    gsem = (g0, g1)
    osem = (o0, o1)
    ssem = (s0, s1, s2, s3)
    wid = lax.axis_index("s") * _NC + lax.axis_index("c")
    irow0 = wid * (_NPW * _S // 128)
    ebase = wid * (_NPW * _S)

    def stage(c, b4):
        pltpu.async_copy(idx2.at[pl.ds(irow0 + c * _IR, _IR)],
                         sidx.at[b4], ssem[b4])
        pltpu.async_copy(maskf.at[pl.ds(ebase + c * _CE, _CE)],
                         smask.at[b4], ssem[b4])

    def swait(b4):
        pltpu.make_async_copy(idx2.at[pl.ds(0, _IR)], sidx.at[b4],
                              ssem[b4]).wait()
        pltpu.make_async_copy(maskf.at[pl.ds(0, _CE)], smask.at[b4],
                              ssem[b4]).wait()

    def fire(b4, b2):
        for q in range(_IR):
            pltpu.async_copy(table.at[sidx.at[b4, q]],
                             rows_v.at[b2, pl.ds(q * 128, 128)], gsem[b2])

    def gwait(b2):
        for q in range(_IR):
            pltpu.make_async_copy(table.at[sidx.at[0, q]],
                                  rows_v.at[b2, pl.ds(q * 128, 128)],
                                  gsem[b2]).wait()

    def owait(b2):
        pltpu.make_async_copy(out_v.at[b2], out4.at[0], osem[b2]).wait()

    def compute(b4, b2):
        def group(g, carry):
            goff = g * (16 * _S)
            mvs = [smask[b4, pl.ds(goff + 16 * w, 16)] for w in range(_S)]
            for ii in range(16):
                for k in range(_H // 16):
                    sl = pl.ds(k * 16, 16)
                    acc = None
                    for s in range(_S):
                        p = 5 * ii + s
                        m = mvs[p // 16][p % 16]
                        term = m * rows_v[b2, goff + p, sl]
                        acc = term if acc is None else acc + term
                    out_v[b2, ii // 2, g,
                          pl.ds((ii % 2) * _H + k * 16, 16)] = acc
            return carry
        lax.fori_loop(0, _CHN // 16, group, 0)

    for c in range(4):
        stage(c, c)
    swait(0)
    fire(0, 0)
    swait(1)
    fire(1, 1)

    def step(t, carry):
        for bb in range(4):
            j = 4 * t + bb
            b2 = bb % 2
            gwait(b2)

            @pl.when(j >= 2)
            def _():
                owait(b2)

            compute(bb, b2)
            pltpu.async_copy(out_v.at[b2], out4.at[wid * _NCH + j],
                             osem[b2])

            @pl.when(j + 2 < _NCH)
            def _():
                swait((bb + 2) % 4)
                fire((bb + 2) % 4, b2)

            @pl.when(j + 4 < _NCH)
            def _():
                stage(j + 4, bb)
        return carry

    lax.fori_loop(0, _NCH // 4, step, 0)
    owait(0)
    owait(1)


@functools.cache
def _sc_gather_fn():
    return pl.kernel(
        _sc_body,
        out_type=jax.ShapeDtypeStruct((_ROWS // 8, _F // 128, 8, 128),
                                      jnp.float32),
        mesh=plsc.VectorSubcoreMesh(core_axis_name="c", subcore_axis_name="s",
                                    num_cores=_NC, num_subcores=_NS),
        scratch_types=[
            pltpu.VMEM((4, _IR, 128), jnp.int32),
            pltpu.VMEM((4, _CE), jnp.float32),
            pltpu.VMEM((2, _CE, _H), jnp.float32),
            pltpu.VMEM((2, _F // 128, 8, 128), jnp.float32),
            pltpu.SemaphoreType.DMA,
            pltpu.SemaphoreType.DMA,
            pltpu.SemaphoreType.DMA,
            pltpu.SemaphoreType.DMA,
            pltpu.SemaphoreType.DMA,
            pltpu.SemaphoreType.DMA,
            pltpu.SemaphoreType.DMA,
            pltpu.SemaphoreType.DMA,
        ],
        compiler_params=pltpu.CompilerParams(use_tc_tiling_on_sc=False),
    )


def _tc_body(x_ref, pc_ref, wl_ref, wp_ref, lw_ref, lb_ref, g_ref, bb_ref,
             o_ref):
    hp = lax.Precision.HIGHEST
    pc = pc_ref[...]
    lw = lw_ref[...]
    wl = wl_ref[...]
    d = wp_ref[...] - wl
    f_id = lax.broadcasted_iota(jnp.int32, (_F, _N), 0)
    n_id = lax.broadcasted_iota(jnp.int32, (_F, _N), 1)
    sel = jnp.where(f_id // _H == n_id, 1.0, 0.0)
    a_t = lax.dot_general(lw * d, sel, (((1,), (0,)), ((), ())),
                          precision=hp, preferred_element_type=jnp.float32)
    c = lax.dot_general(wl, lw, (((1,), (1,)), ((), ())),
                        precision=hp, preferred_element_type=jnp.float32)
    y = lax.dot_general(pc, a_t, (((1,), (1,)), ((), ())),
                        precision=hp, preferred_element_type=jnp.float32)
    for ct in range(_F // 128):
        xc = x_ref[:, ct].reshape(_BR, 128)
        lwc = lw[:, ct * 128:(ct + 1) * 128]
        y = y + lax.dot_general(xc, lwc, (((1,), (1,)), ((), ())),
                                precision=hp,
                                preferred_element_type=jnp.float32)
    y = y + c + lb_ref[...]
    mu = jnp.mean(y, axis=1, keepdims=True)
    yc = y - mu
    var = jnp.mean(yc * yc, axis=1, keepdims=True)
    o_ref[...] = yc * lax.rsqrt(var + 1e-5) * g_ref[...] + bb_ref[...]


_BR = 512

_tc_mix = pl.pallas_call(
    _tc_body,
    out_shape=jax.ShapeDtypeStruct((_ROWS, _H), jnp.float32),
    grid=(_ROWS // _BR,),
    in_specs=[
        pl.BlockSpec((_BR // 8, _F // 128, 8, 128), lambda i: (i, 0, 0, 0)),
        pl.BlockSpec((_BR, _N), lambda i: (i, 0)),
        pl.BlockSpec((1, _F), lambda i: (0, 0)),
        pl.BlockSpec((1, _F), lambda i: (0, 0)),
        pl.BlockSpec((_H, _F), lambda i: (0, 0)),
        pl.BlockSpec((1, _H), lambda i: (0, 0)),
        pl.BlockSpec((1, _H), lambda i: (0, 0)),
        pl.BlockSpec((1, _H), lambda i: (0, 0)),
    ],
    out_specs=pl.BlockSpec((_BR, _H), lambda i: (i, 0)),
)


def kernel(src, pos_coeff, src_subtoken_mask, tok_embedding, W_level,
           W_parent, lin_W, lin_b, ln_g, ln_b):
    x4 = _sc_gather_fn()(tok_embedding, src.reshape(_NT * _S // 128, 128),
                         src_subtoken_mask.reshape(-1))
    out = _tc_mix(x4, pos_coeff.reshape(_ROWS, _N),
                  W_level.reshape(1, _F), W_parent.reshape(1, _F), lin_W,
                  lin_b.reshape(1, _H), ln_g.reshape(1, _H),
                  ln_b.reshape(1, _H))
    return out.reshape(_B, _L, _H)


# speedup vs baseline: 1.2309x; 1.1906x over previous
import functools

import jax
import jax.numpy as jnp
from jax import lax
from jax.experimental import pallas as pl
from jax.experimental.pallas import tpu as pltpu
from jax.experimental.pallas import tpu_sc as plsc

_B, _L, _N, _S, _H = 16, 512, 16, 5, 64
_NT = _B * _L * _N
_NC, _NS = 2, 16
_NW = _NC * _NS
_NPW = _NT // _NW
_CH = 64
_NCHUNK = _NPW // _CH
_ROWS = _NT // _N
_F = _N * _H


_CHN = 128
_CE = _CHN * _S
_NCH = _NPW // _CHN
_NG = _CHN // 16
_GE = 16 * _S


def _sc_body(table, idx2, maskf, out4, sidx, smask, rows_v, out_v,
             g0, g1, o0, o1, s0, s1, s2, s3):
    gsem = (g0, g1)
    osem = (o0, o1)
    ssem = (s0, s1, s2, s3)
    wid = lax.axis_index("s") * _NC + lax.axis_index("c")
    irow0 = wid * (_NPW * _S // _GE)
    ebase = wid * (_NPW * _S)

    def stage(c, b4):
        pltpu.async_copy(idx2.at[pl.ds(irow0 + c * _NG, _NG)],
                         sidx.at[b4], ssem[b4])
        pltpu.async_copy(maskf.at[pl.ds(ebase + c * _CE, _CE)],
                         smask.at[b4], ssem[b4])

    def swait(b4):
        pltpu.make_async_copy(idx2.at[pl.ds(0, _NG)], sidx.at[b4],
                              ssem[b4]).wait()
        pltpu.make_async_copy(maskf.at[pl.ds(0, _CE)], smask.at[b4],
                              ssem[b4]).wait()

    def fire(b4, b2):
        for q in range(_NG):
            pltpu.async_copy(table.at[sidx.at[b4, q]],
                             rows_v.at[b2, q], gsem[b2])

    def gwait(b2):
        for q in range(_NG):
            pltpu.make_async_copy(table.at[sidx.at[0, q]],
                                  rows_v.at[b2, q], gsem[b2]).wait()

    def owait(b2):
        pltpu.make_async_copy(out_v.at[b2], out4.at[0], osem[b2]).wait()

    def compute(b4, b2):
        def group(g, carry):
            goff = g * _GE
            mvs = [smask[b4, pl.ds(goff + 16 * w, 16)] for w in range(_S)]
            for ii in range(16):
                for k in range(_H // 16):
                    sl = pl.ds(k * 16, 16)
                    acc = None
                    for s in range(_S):
                        p = 5 * ii + s
                        m = mvs[p // 16][p % 16]
                        term = m * rows_v[b2, g, p, sl]
                        acc = term if acc is None else acc + term
                    out_v[b2, ii // 2, g,
                          pl.ds((ii % 2) * _H + k * 16, 16)] = acc
            return carry
        lax.fori_loop(0, _NG, group, 0)

    for c in range(4):
        stage(c, c)
    swait(0)
    fire(0, 0)
    swait(1)
    fire(1, 1)

    def step(t, carry):
        for bb in range(4):
            j = 4 * t + bb
            b2 = bb % 2
            gwait(b2)

            @pl.when(j >= 2)
            def _():
                owait(b2)

            compute(bb, b2)
            pltpu.async_copy(out_v.at[b2], out4.at[wid * _NCH + j],
                             osem[b2])

            @pl.when(j + 2 < _NCH)
            def _():
                swait((bb + 2) % 4)
                fire((bb + 2) % 4, b2)

            @pl.when(j + 4 < _NCH)
            def _():
                stage(j + 4, bb)
        return carry

    lax.fori_loop(0, _NCH // 4, step, 0)
    owait(0)
    owait(1)


@functools.cache
def _sc_gather_fn():
    return pl.kernel(
        _sc_body,
        out_type=jax.ShapeDtypeStruct((_ROWS // 8, _F // 128, 8, 128),
                                      jnp.float32),
        mesh=plsc.VectorSubcoreMesh(core_axis_name="c", subcore_axis_name="s",
                                    num_cores=_NC, num_subcores=_NS),
        scratch_types=[
            pltpu.VMEM((4, _NG, _GE), jnp.int32),
            pltpu.VMEM((4, _CE), jnp.float32),
            pltpu.VMEM((2, _NG, _GE, _H), jnp.float32),
            pltpu.VMEM((2, _F // 128, 8, 128), jnp.float32),
            pltpu.SemaphoreType.DMA,
            pltpu.SemaphoreType.DMA,
            pltpu.SemaphoreType.DMA,
            pltpu.SemaphoreType.DMA,
            pltpu.SemaphoreType.DMA,
            pltpu.SemaphoreType.DMA,
            pltpu.SemaphoreType.DMA,
            pltpu.SemaphoreType.DMA,
        ],
        compiler_params=pltpu.CompilerParams(use_tc_tiling_on_sc=False),
    )


def _tc_body(x_ref, pc_ref, wl_ref, wp_ref, lw_ref, lb_ref, g_ref, bb_ref,
             o_ref):
    hp = lax.Precision.HIGHEST
    pc = pc_ref[...]
    lw = lw_ref[...]
    wl = wl_ref[...]
    d = wp_ref[...] - wl
    f_id = lax.broadcasted_iota(jnp.int32, (_F, _N), 0)
    n_id = lax.broadcasted_iota(jnp.int32, (_F, _N), 1)
    sel = jnp.where(f_id // _H == n_id, 1.0, 0.0)
    a_t = lax.dot_general(lw * d, sel, (((1,), (0,)), ((), ())),
                          precision=hp, preferred_element_type=jnp.float32)
    c = lax.dot_general(wl, lw, (((1,), (1,)), ((), ())),
                        precision=hp, preferred_element_type=jnp.float32)
    y = lax.dot_general(pc, a_t, (((1,), (1,)), ((), ())),
                        precision=hp, preferred_element_type=jnp.float32)
    for ct in range(_F // 128):
        xc = x_ref[:, ct].reshape(_BR, 128)
        lwc = lw[:, ct * 128:(ct + 1) * 128]
        y = y + lax.dot_general(xc, lwc, (((1,), (1,)), ((), ())),
                                precision=hp,
                                preferred_element_type=jnp.float32)
    y = y + c + lb_ref[...]
    mu = jnp.mean(y, axis=1, keepdims=True)
    yc = y - mu
    var = jnp.mean(yc * yc, axis=1, keepdims=True)
    o_ref[...] = yc * lax.rsqrt(var + 1e-5) * g_ref[...] + bb_ref[...]


_BR = 512

_tc_mix = pl.pallas_call(
    _tc_body,
    out_shape=jax.ShapeDtypeStruct((_ROWS, _H), jnp.float32),
    grid=(_ROWS // _BR,),
    in_specs=[
        pl.BlockSpec((_BR // 8, _F // 128, 8, 128), lambda i: (i, 0, 0, 0)),
        pl.BlockSpec((_BR, _N), lambda i: (i, 0)),
        pl.BlockSpec((1, _F), lambda i: (0, 0)),
        pl.BlockSpec((1, _F), lambda i: (0, 0)),
        pl.BlockSpec((_H, _F), lambda i: (0, 0)),
        pl.BlockSpec((1, _H), lambda i: (0, 0)),
        pl.BlockSpec((1, _H), lambda i: (0, 0)),
        pl.BlockSpec((1, _H), lambda i: (0, 0)),
    ],
    out_specs=pl.BlockSpec((_BR, _H), lambda i: (i, 0)),
)


def kernel(src, pos_coeff, src_subtoken_mask, tok_embedding, W_level,
           W_parent, lin_W, lin_b, ln_g, ln_b):
    x4 = _sc_gather_fn()(tok_embedding, src.reshape(_NT * _S // _GE, _GE),
                         src_subtoken_mask.reshape(-1))
    out = _tc_mix(x4, pos_coeff.reshape(_ROWS, _N),
                  W_level.reshape(1, _F), W_parent.reshape(1, _F), lin_W,
                  lin_b.reshape(1, _H), ln_g.reshape(1, _H),
                  ln_b.reshape(1, _H))
    return out.reshape(_B, _L, _H)


# speedup vs baseline: 1.5399x; 1.2510x over previous
import functools

import jax
import jax.numpy as jnp
from jax import lax
from jax.experimental import pallas as pl
from jax.experimental.pallas import tpu as pltpu
from jax.experimental.pallas import tpu_sc as plsc

_B, _L, _N, _S, _H = 16, 512, 16, 5, 64
_NT = _B * _L * _N
_NC, _NS = 2, 16
_NW = _NC * _NS
_NPW = _NT // _NW
_CH = 64
_NCHUNK = _NPW // _CH
_ROWS = _NT // _N
_F = _N * _H


_CHN = 128
_CE = _CHN * _S
_NCH = _NPW // _CHN
_NG = _CHN // 16
_GE = 16 * _S


def _sc_body(table, idx2, maskf, out4, sidx, smask, rows_v, out_v,
             g0, g1, o0, o1, s0, s1, s2, s3):
    gsem = (g0, g1)
    osem = (o0, o1)
    ssem = (s0, s1, s2, s3)
    wid = lax.axis_index("s") * _NC + lax.axis_index("c")
    irow0 = wid * (_NPW * _S // _GE)

    def stage(c, b4):
        pltpu.async_copy(idx2.at[pl.ds(irow0 + c * _NG, _NG)],
                         sidx.at[b4], ssem[b4])
        pltpu.async_copy(maskf.at[pl.ds(irow0 + c * _NG, _NG)],
                         smask.at[b4], ssem[b4])

    def swait(b4):
        pltpu.make_async_copy(idx2.at[pl.ds(0, _NG)], sidx.at[b4],
                              ssem[b4]).wait()
        pltpu.make_async_copy(maskf.at[pl.ds(0, _NG)], smask.at[b4],
                              ssem[b4]).wait()

    def fire(b4, b2):
        for q in range(_NG):
            pltpu.async_copy(table.at[sidx.at[b4, q]],
                             rows_v.at[b2, q], gsem[b2])

    def gwait(b2):
        for q in range(_NG):
            pltpu.make_async_copy(table.at[sidx.at[0, q]],
                                  rows_v.at[b2, q], gsem[b2]).wait()

    def owait(b2):
        pltpu.make_async_copy(out_v.at[b2], out4.at[0], osem[b2]).wait()

    def compute(b4, b2):
        def group(g, carry):
            mvs = [smask[b4, g, pl.ds(16 * w, 16)] for w in range(_S)]
            for ii in range(16):
                ms = [mvs[(5 * ii + s) // 16][(5 * ii + s) % 16]
                      for s in range(_S)]
                for k in range(_H // 16):
                    sl = pl.ds(k * 16, 16)
                    acc = None
                    for s in range(_S):
                        term = ms[s] * rows_v[b2, g, 5 * ii + s, sl]
                        acc = term if acc is None else acc + term
                    out_v[b2, ii // 2, g,
                          pl.ds((ii % 2) * _H + k * 16, 16)] = acc
            return carry
        lax.fori_loop(0, _NG, group, 0)

    for c in range(4):
        stage(c, c)
    swait(0)
    fire(0, 0)
    swait(1)
    fire(1, 1)

    def step(t, carry):
        for bb in range(4):
            j = 4 * t + bb
            b2 = bb % 2
            gwait(b2)

            @pl.when(j >= 2)
            def _():
                owait(b2)

            compute(bb, b2)
            pltpu.async_copy(out_v.at[b2], out4.at[wid * _NCH + j],
                             osem[b2])

            @pl.when(j + 2 < _NCH)
            def _():
                swait((bb + 2) % 4)
                fire((bb + 2) % 4, b2)

            @pl.when(j + 4 < _NCH)
            def _():
                stage(j + 4, bb)
        return carry

    lax.fori_loop(0, _NCH // 4, step, 0)
    owait(0)
    owait(1)


@functools.cache
def _sc_gather_fn():
    return pl.kernel(
        _sc_body,
        out_type=jax.ShapeDtypeStruct((_ROWS // 8, _F // 128, 8, 128),
                                      jnp.float32),
        mesh=plsc.VectorSubcoreMesh(core_axis_name="c", subcore_axis_name="s",
                                    num_cores=_NC, num_subcores=_NS),
        scratch_types=[
            pltpu.VMEM((4, _NG, _GE), jnp.int32),
            pltpu.VMEM((4, _NG, _GE), jnp.float32),
            pltpu.VMEM((2, _NG, _GE, _H), jnp.float32),
            pltpu.VMEM((2, _F // 128, 8, 128), jnp.float32),
            pltpu.SemaphoreType.DMA,
            pltpu.SemaphoreType.DMA,
            pltpu.SemaphoreType.DMA,
            pltpu.SemaphoreType.DMA,
            pltpu.SemaphoreType.DMA,
            pltpu.SemaphoreType.DMA,
            pltpu.SemaphoreType.DMA,
            pltpu.SemaphoreType.DMA,
        ],
        compiler_params=pltpu.CompilerParams(use_tc_tiling_on_sc=False),
    )


def _tc_body(x_ref, pc_ref, wl_ref, wp_ref, lw_ref, lb_ref, g_ref, bb_ref,
             o_ref):
    hp = lax.Precision.HIGHEST
    pc = pc_ref[...].reshape(_BR, _N)
    lw = lw_ref[...]
    wl = wl_ref[...]
    d = wp_ref[...] - wl
    f_id = lax.broadcasted_iota(jnp.int32, (_F, _N), 0)
    n_id = lax.broadcasted_iota(jnp.int32, (_F, _N), 1)
    sel = jnp.where(f_id // _H == n_id, 1.0, 0.0)
    a_t = lax.dot_general(lw * d, sel, (((1,), (0,)), ((), ())),
                          precision=hp, preferred_element_type=jnp.float32)
    c = lax.dot_general(wl, lw, (((1,), (1,)), ((), ())),
                        precision=hp, preferred_element_type=jnp.float32)
    y = lax.dot_general(pc, a_t, (((1,), (1,)), ((), ())),
                        precision=hp, preferred_element_type=jnp.float32)
    for ct in range(_F // 128):
        xc = x_ref[:, ct].reshape(_BR, 128)
        lwc = lw[:, ct * 128:(ct + 1) * 128]
        y = y + lax.dot_general(xc, lwc, (((1,), (1,)), ((), ())),
                                precision=hp,
                                preferred_element_type=jnp.float32)
    y = y + c + lb_ref[...]
    mu = jnp.mean(y, axis=1, keepdims=True)
    yc = y - mu
    var = jnp.mean(yc * yc, axis=1, keepdims=True)
    res = yc * lax.rsqrt(var + 1e-5) * g_ref[...] + bb_ref[...]
    o_ref[...] = res.reshape(1, _BR, _H)


_BR = 512

_tc_mix = pl.pallas_call(
    _tc_body,
    out_shape=jax.ShapeDtypeStruct((_B, _L, _H), jnp.float32),
    grid=(_ROWS // _BR,),
    in_specs=[
        pl.BlockSpec((_BR // 8, _F // 128, 8, 128), lambda i: (i, 0, 0, 0)),
        pl.BlockSpec((1, _L, _N), lambda i: (i, 0, 0)),
        pl.BlockSpec((1, _F), lambda i: (0, 0)),
        pl.BlockSpec((1, _F), lambda i: (0, 0)),
        pl.BlockSpec((_H, _F), lambda i: (0, 0)),
        pl.BlockSpec((1, _H), lambda i: (0, 0)),
        pl.BlockSpec((1, _H), lambda i: (0, 0)),
        pl.BlockSpec((1, _H), lambda i: (0, 0)),
    ],
    out_specs=pl.BlockSpec((1, _L, _H), lambda i: (i, 0, 0)),
)


def kernel(src, pos_coeff, src_subtoken_mask, tok_embedding, W_level,
           W_parent, lin_W, lin_b, ln_g, ln_b):
    x4 = _sc_gather_fn()(tok_embedding, src.reshape(_NT * _S // _GE, _GE),
                         src_subtoken_mask.reshape(_NT * _S // _GE, _GE))
    return _tc_mix(x4, pos_coeff,
                   W_level.reshape(1, _F), W_parent.reshape(1, _F), lin_W,
                   lin_b.reshape(1, _H), ln_g.reshape(1, _H),
                   ln_b.reshape(1, _H))


# speedup vs baseline: 1.6633x; 1.0801x over previous
import functools

import jax
import jax.numpy as jnp
from jax import lax
from jax.experimental import pallas as pl
from jax.experimental.pallas import tpu as pltpu
from jax.experimental.pallas import tpu_sc as plsc

_B, _L, _N, _S, _H = 16, 512, 16, 5, 64
_NT = _B * _L * _N
_NC, _NS = 2, 16
_NW = _NC * _NS
_NPW = _NT // _NW
_CH = 64
_NCHUNK = _NPW // _CH
_ROWS = _NT // _N
_F = _N * _H


_CHN = 128
_CE = _CHN * _S
_NCH = _NPW // _CHN
_NG = _CHN // 16
_GE = 16 * _S


def _sc_body(table, idx2, maskf, out4, sidx, smask, rows_v, out_v,
             g0, g1, o0, o1, s0, s1, s2, s3):
    gsem = (g0, g1)
    osem = (o0, o1)
    ssem = (s0, s1, s2, s3)
    wid = lax.axis_index("s") * _NC + lax.axis_index("c")
    irow0 = wid * (_NPW * _S // _GE)

    def stage(c, b4):
        pltpu.async_copy(idx2.at[pl.ds(irow0 + c * _NG, _NG)],
                         sidx.at[b4], ssem[b4])
        pltpu.async_copy(maskf.at[pl.ds(irow0 + c * _NG, _NG)],
                         smask.at[b4], ssem[b4])

    def swait(b4):
        pltpu.make_async_copy(idx2.at[pl.ds(0, _NG)], sidx.at[b4],
                              ssem[b4]).wait()
        pltpu.make_async_copy(maskf.at[pl.ds(0, _NG)], smask.at[b4],
                              ssem[b4]).wait()

    def fire(b4, b2):
        for q in range(_NG):
            pltpu.async_copy(table.at[sidx.at[b4, q]],
                             rows_v.at[b2, q], gsem[b2])

    def gwait(b2):
        for q in range(_NG):
            pltpu.make_async_copy(table.at[sidx.at[0, q]],
                                  rows_v.at[b2, q], gsem[b2]).wait()

    def owait(b2):
        pltpu.make_async_copy(out_v.at[b2], out4.at[0], osem[b2]).wait()

    def compute(b4, b2):
        @plsc.parallel_loop(0, _NG)
        def group(g):
            mvs = [smask[b4, g, pl.ds(16 * w, 16)] for w in range(_S)]
            for ii in range(16):
                ms = [mvs[(5 * ii + s) // 16][(5 * ii + s) % 16]
                      for s in range(_S)]
                for k in range(_H // 16):
                    sl = pl.ds(k * 16, 16)
                    acc = None
                    for s in range(_S):
                        term = ms[s] * rows_v[b2, g, 5 * ii + s, sl]
                        acc = term if acc is None else acc + term
                    out_v[b2, ii // 2, g,
                          pl.ds((ii % 2) * _H + k * 16, 16)] = acc

    for c in range(4):
        stage(c, c)
    swait(0)
    fire(0, 0)
    swait(1)
    fire(1, 1)

    def step(t, carry):
        for bb in range(4):
            j = 4 * t + bb
            b2 = bb % 2
            gwait(b2)

            @pl.when(j >= 2)
            def _():
                owait(b2)

            compute(bb, b2)
            pltpu.async_copy(out_v.at[b2], out4.at[wid * _NCH + j],
                             osem[b2])

            @pl.when(j + 2 < _NCH)
            def _():
                swait((bb + 2) % 4)
                fire((bb + 2) % 4, b2)

            @pl.when(j + 4 < _NCH)
            def _():
                stage(j + 4, bb)
        return carry

    lax.fori_loop(0, _NCH // 4, step, 0)
    owait(0)
    owait(1)


@functools.cache
def _sc_gather_fn():
    return pl.kernel(
        _sc_body,
        out_type=jax.ShapeDtypeStruct((_ROWS // 8, _F // 128, 8, 128),
                                      jnp.float32),
        mesh=plsc.VectorSubcoreMesh(core_axis_name="c", subcore_axis_name="s",
                                    num_cores=_NC, num_subcores=_NS),
        scratch_types=[
            pltpu.VMEM((4, _NG, _GE), jnp.int32),
            pltpu.VMEM((4, _NG, _GE), jnp.float32),
            pltpu.VMEM((2, _NG, _GE, _H), jnp.float32),
            pltpu.VMEM((2, _F // 128, 8, 128), jnp.float32),
            pltpu.SemaphoreType.DMA,
            pltpu.SemaphoreType.DMA,
            pltpu.SemaphoreType.DMA,
            pltpu.SemaphoreType.DMA,
            pltpu.SemaphoreType.DMA,
            pltpu.SemaphoreType.DMA,
            pltpu.SemaphoreType.DMA,
            pltpu.SemaphoreType.DMA,
        ],
        compiler_params=pltpu.CompilerParams(use_tc_tiling_on_sc=False),
    )


def _tc_body(x_ref, pc_ref, wl_ref, wp_ref, lw_ref, lb_ref, g_ref, bb_ref,
             o_ref):
    hp = lax.Precision.DEFAULT
    pc = pc_ref[...].reshape(_BR, _N)
    lw = lw_ref[...]
    wl = wl_ref[...]
    d = wp_ref[...] - wl
    f_id = lax.broadcasted_iota(jnp.int32, (_F, _N), 0)
    n_id = lax.broadcasted_iota(jnp.int32, (_F, _N), 1)
    sel = jnp.where(f_id // _H == n_id, 1.0, 0.0)
    a_t = lax.dot_general(lw * d, sel, (((1,), (0,)), ((), ())),
                          precision=hp, preferred_element_type=jnp.float32)
    c = lax.dot_general(wl, lw, (((1,), (1,)), ((), ())),
                        precision=hp, preferred_element_type=jnp.float32)
    y = lax.dot_general(pc, a_t, (((1,), (1,)), ((), ())),
                        precision=hp, preferred_element_type=jnp.float32)
    for ct in range(_F // 128):
        xc = x_ref[:, ct].reshape(_BR, 128)
        lwc = lw[:, ct * 128:(ct + 1) * 128]
        y = y + lax.dot_general(xc, lwc, (((1,), (1,)), ((), ())),
                                precision=hp,
                                preferred_element_type=jnp.float32)
    y = y + c + lb_ref[...]
    mu = jnp.mean(y, axis=1, keepdims=True)
    yc = y - mu
    var = jnp.mean(yc * yc, axis=1, keepdims=True)
    res = yc * lax.rsqrt(var + 1e-5) * g_ref[...] + bb_ref[...]
    o_ref[...] = res.reshape(1, _BR, _H)


_BR = 512

_tc_mix = pl.pallas_call(
    _tc_body,
    out_shape=jax.ShapeDtypeStruct((_B, _L, _H), jnp.float32),
    grid=(_ROWS // _BR,),
    in_specs=[
        pl.BlockSpec((_BR // 8, _F // 128, 8, 128), lambda i: (i, 0, 0, 0)),
        pl.BlockSpec((1, _L, _N), lambda i: (i, 0, 0)),
        pl.BlockSpec((1, _F), lambda i: (0, 0)),
        pl.BlockSpec((1, _F), lambda i: (0, 0)),
        pl.BlockSpec((_H, _F), lambda i: (0, 0)),
        pl.BlockSpec((1, _H), lambda i: (0, 0)),
        pl.BlockSpec((1, _H), lambda i: (0, 0)),
        pl.BlockSpec((1, _H), lambda i: (0, 0)),
    ],
    out_specs=pl.BlockSpec((1, _L, _H), lambda i: (i, 0, 0)),
)


def kernel(src, pos_coeff, src_subtoken_mask, tok_embedding, W_level,
           W_parent, lin_W, lin_b, ln_g, ln_b):
    x4 = _sc_gather_fn()(tok_embedding, src.reshape(_NT * _S // _GE, _GE),
                         src_subtoken_mask.reshape(_NT * _S // _GE, _GE))
    return _tc_mix(x4, pos_coeff,
                   W_level.reshape(1, _F), W_parent.reshape(1, _F), lin_W,
                   lin_b.reshape(1, _H), ln_g.reshape(1, _H),
                   ln_b.reshape(1, _H))
